# baseline (speedup 1.0000x reference)
import dataclasses
import functools

import jax
import jax.numpy as jnp
from jax import lax
from jax.experimental import pallas as pl
from jax.experimental.pallas import tpu as pltpu
from jax.experimental.pallas import tpu_sc as plsc

N_NODES = 10000
N_DST0 = 5000
N_DST1 = 2500
E0 = 320000
E1 = 160000
D_IN = 128
D_HID = 16
D_OUT = 64

NC, NS = 2, 16
NW = NC * NS
CH = 128
NCH0 = 80
NCH1 = 40

H_S0 = 10112
H_N0 = 5120
H_N1 = 2560

P_ROWS = 10112
A0_ROWS = 5120
H_ROWS = 5120
A1_ROWS = 2560

def _mesh():
    return plsc.VectorSubcoreMesh(core_axis_name="c", subcore_axis_name="s",
                                  num_cores=NC, num_subcores=NS)


def _sc_params(vector_ops=False):
    cp = pltpu.CompilerParams(use_tc_tiling_on_sc=False)
    if vector_ops and "needs_layout_passes" in pltpu.CompilerParams.__dataclass_fields__:
        cp = dataclasses.replace(cp, needs_layout_passes=False)
    return cp
_f32 = jnp.float32


def _zero_rows(z_v, n):
    @pl.loop(0, n)
    def _(i):
        z_v[i, :] = jnp.zeros((16,), _f32)


def _hist_body(s0, d0, s1, d1, o0, o1, o2, o3,
               g0, g1, g2, g3, idx_v, sm0, sm1, sm2, sm3):
    c = lax.axis_index("c")
    s = lax.axis_index("s")
    wid = s * NC + c
    ones = jnp.ones((16,), _f32)

    segs = ((s0, g0, H_S0, NCH0, 0, o0, sm0),
            (d0, g1, H_N0, NCH0, NCH0, o1, sm1),
            (s1, g2, H_N0, NCH1, 2 * NCH0, o2, sm2),
            (d1, g3, H_N1, NCH1, 2 * NCH0 + NCH1, o3, sm3))
    for arr, _, _, nch, off, _, sm in segs:
        pltpu.async_copy(arr.at[wid], idx_v.at[pl.ds(off, nch)], sm)

    for arr, hv, hn, nch, off, o, sm in segs:
        @pl.loop(0, hn, step=128)
        def _(i, hv=hv):
            for k in range(8):
                hv[pl.ds(i + k * 16, 16)] = jnp.zeros((16,), _f32)

        pltpu.make_async_copy(arr.at[wid], idx_v.at[pl.ds(off, nch)], sm).wait()

        @pl.loop(off, off + nch, unroll=2)
        def _(j, hv=hv):
            for k in range(CH // 16):
                plsc.addupdate_scatter(hv, [idx_v[j, pl.ds(k * 16, 16)]], ones)

        pltpu.async_copy(hv, o.at[wid], sm)
    for arr, hv, hn, nch, off, o, sm in segs:
        pltpu.make_async_copy(hv, o.at[wid], sm).wait()


@jax.jit
def _hist(s0p, d0p, s1p, d1p):
    f = pl.kernel(
        _hist_body,
        out_type=(jax.ShapeDtypeStruct((NW, H_S0), _f32),
                  jax.ShapeDtypeStruct((NW, H_N0), _f32),
                  jax.ShapeDtypeStruct((NW, H_N0), _f32),
                  jax.ShapeDtypeStruct((NW, H_N1), _f32)),
        mesh=_mesh(),
        compiler_params=_sc_params(vector_ops=True),
        scratch_types=[
            pltpu.VMEM((H_S0,), _f32),
            pltpu.VMEM((H_N0,), _f32),
            pltpu.VMEM((H_N0,), _f32),
            pltpu.VMEM((H_N1,), _f32),
            pltpu.VMEM((2 * (NCH0 + NCH1), CH), jnp.int32),
            pltpu.SemaphoreType.DMA,
            pltpu.SemaphoreType.DMA,
            pltpu.SemaphoreType.DMA,
            pltpu.SemaphoreType.DMA,
        ],
    )
    return f(s0p, d0p, s1p, d1p)


def _edge_body(nch, tab_rows, acc_rows,
               feat, src, dst, out, acc_sh, src_v, dst_v,
               rows0, rows1, z_v, gs0, gs1, ss0, ss1):
    c = lax.axis_index("c")
    s = lax.axis_index("s")
    wid = s * NC + c
    ka = acc_rows // 16

    _zero_rows(z_v, ka)
    pltpu.async_copy(z_v.at[pl.ds(0, ka)], acc_sh.at[pl.ds(s * ka, ka)], gs1)
    pltpu.async_copy(src.at[wid], src_v, ss0)
    pltpu.async_copy(dst.at[wid], dst_v, ss1)
    pltpu.make_async_copy(z_v.at[pl.ds(0, ka)], acc_sh.at[pl.ds(s * ka, ka)], gs1).wait()
    pltpu.make_async_copy(src.at[wid], src_v, ss0).wait()
    pltpu.make_async_copy(dst.at[wid], dst_v, ss1).wait()
    plsc.subcore_barrier()

    pltpu.async_copy(feat.at[src_v.at[0]], rows0, gs0)
    pltpu.async_copy(feat.at[src_v.at[1]], rows1, gs1)

    @pl.loop(0, nch, step=2)
    def _(j):
        pltpu.make_async_copy(feat.at[src_v.at[0]], rows0, gs0).wait()
        pltpu.async_copy(rows0, acc_sh.at[dst_v.at[j]], ss0, add=True)
        pltpu.make_async_copy(feat.at[src_v.at[0]], rows1, gs1).wait()
        pltpu.async_copy(rows1, acc_sh.at[dst_v.at[j + 1]], ss1, add=True)
        pltpu.make_async_copy(rows0, acc_sh.at[dst_v.at[0]], ss0).wait()

        @pl.when(j + 2 < nch)
        def _():
            pltpu.async_copy(feat.at[src_v.at[j + 2]], rows0, gs0)

        pltpu.make_async_copy(rows1, acc_sh.at[dst_v.at[0]], ss1).wait()

        @pl.when(j + 3 < nch)
        def _():
            pltpu.async_copy(feat.at[src_v.at[j + 3]], rows1, gs1)

    plsc.subcore_barrier()
    pltpu.sync_copy(acc_sh.at[pl.ds(s * ka, ka)], out.at[c].at[pl.ds(s * ka, ka)])


def _make_edge(nch, tab_rows, acc_rows):
    @jax.jit
    def f(feat, src, dst):
        body = functools.partial(_edge_body, nch, tab_rows, acc_rows)
        k = pl.kernel(
            body,
            out_type=jax.ShapeDtypeStruct((NC, acc_rows, D_HID), _f32),
            mesh=_mesh(),
            compiler_params=_sc_params(),
            scratch_types=[
                pltpu.VMEM_SHARED((acc_rows, D_HID), _f32),
                pltpu.VMEM((nch, CH), jnp.int32),
                pltpu.VMEM((nch, CH), jnp.int32),
                pltpu.VMEM((CH, D_HID), _f32),
                pltpu.VMEM((CH, D_HID), _f32),
                pltpu.VMEM((acc_rows // 16, D_HID), _f32),
                pltpu.SemaphoreType.DMA,
                pltpu.SemaphoreType.DMA,
                pltpu.SemaphoreType.DMA,
                pltpu.SemaphoreType.DMA,
            ],
        )
        return k(feat, src, dst)
    return f


_edge0 = _make_edge(NCH0, P_ROWS, A0_ROWS)
_edge1 = _make_edge(NCH1, H_ROWS, A1_ROWS)


def _mm_body(x_ref, w_ref, o_ref):
    o_ref[...] = jnp.dot(x_ref[...], w_ref[...], preferred_element_type=_f32)


@jax.jit
def _mm(x, w):
    return pl.pallas_call(
        _mm_body,
        out_shape=jax.ShapeDtypeStruct((N_NODES, D_HID), _f32),
    )(x, w)


def _scale_body(d_ref, p_ref, o_ref):
    d = jnp.sum(d_ref[...], axis=0)[:N_NODES]
    r = lax.rsqrt(jnp.maximum(d, 1.0))
    o_ref[pl.ds(0, N_NODES), :] = p_ref[...] * r[:, None]
    o_ref[pl.ds(N_NODES, P_ROWS - N_NODES), :] = jnp.zeros(
        (P_ROWS - N_NODES, D_HID), _f32)


@jax.jit
def _proj(deg, p0):
    return pl.pallas_call(
        _scale_body,
        out_shape=jax.ShapeDtypeStruct((P_ROWS, D_HID), _f32),
    )(deg, p0)


def _mid_body(p_ref, din_ref, dout_ref, b_ref, o_ref):
    h = p_ref[0, :N_DST0] + p_ref[1, :N_DST0]
    rin = lax.rsqrt(jnp.maximum(jnp.sum(din_ref[...], axis=0)[:N_DST0], 1.0))
    rout = lax.rsqrt(jnp.maximum(jnp.sum(dout_ref[...], axis=0)[:N_DST0], 1.0))
    h = jnp.maximum(h * rin[:, None] + b_ref[...], 0.0) * rout[:, None]
    o_ref[pl.ds(0, N_DST0), :] = h
    o_ref[pl.ds(N_DST0, H_ROWS - N_DST0), :] = jnp.zeros(
        (H_ROWS - N_DST0, D_HID), _f32)


@jax.jit
def _mid(p, din, dout, b):
    return pl.pallas_call(
        _mid_body,
        out_shape=jax.ShapeDtypeStruct((H_ROWS, D_HID), _f32),
    )(p, din, dout, b)


def _out_body(q_ref, din_ref, w_ref, b_ref, o_ref):
    agg = q_ref[0, :N_DST1] + q_ref[1, :N_DST1]
    r = lax.rsqrt(jnp.maximum(jnp.sum(din_ref[...], axis=0)[:N_DST1], 1.0))
    o_ref[...] = jnp.dot(agg, w_ref[...],
                         preferred_element_type=_f32) * r[:, None] + b_ref[...]


@jax.jit
def _outk(q, din, w, b):
    return pl.pallas_call(
        _out_body,
        out_shape=jax.ShapeDtypeStruct((N_DST1, D_OUT), _f32),
    )(q, din, w, b)


def kernel(x, src0, dst0, src1, dst1, W1, b1, W2, b2):
    i32 = jnp.int32
    src0 = src0.astype(i32)
    dst0 = dst0.astype(i32)
    src1 = src1.astype(i32)
    dst1 = dst1.astype(i32)

    pad0 = NW * NCH0 * CH - E0
    s0p = jnp.concatenate([src0, jnp.full((pad0,), N_NODES, i32)]).reshape(NW, NCH0, CH)
    d0p = jnp.concatenate([dst0, jnp.full((pad0,), N_DST0, i32)]).reshape(NW, NCH0, CH)
    pad1 = NW * NCH1 * CH - E1
    s1p = jnp.concatenate([src1, jnp.full((pad1,), N_DST0, i32)]).reshape(NW, NCH1, CH)
    d1p = jnp.concatenate([dst1, jnp.full((pad1,), N_DST1, i32)]).reshape(NW, NCH1, CH)

    p0 = _mm(x, W1)
    h_s0, h_d0, h_s1, h_d1 = _hist(s0p, d0p, s1p, d1p)

    p = _proj(h_s0, p0)
    agg0 = _edge0(p, s0p, d0p)

    hmid = _mid(agg0, h_d0, h_s1, b1.reshape(1, D_HID))
    agg1 = _edge1(hmid, s1p, d1p)

    return _outk(agg1, h_d1, W2, b2.reshape(1, D_OUT))


# speedup vs baseline: 1.4618x; 1.4618x over previous
import dataclasses
import functools

import jax
import jax.numpy as jnp
from jax import lax
from jax.experimental import pallas as pl
from jax.experimental.pallas import tpu as pltpu
from jax.experimental.pallas import tpu_sc as plsc

N_NODES = 10000
N_DST0 = 5000
N_DST1 = 2500
E0 = 320000
E1 = 160000
D_IN = 128
D_HID = 16
D_OUT = 64

NC, NS = 2, 16
NW = NC * NS
CH = 128
NCH0 = 80
NCH1 = 40

H_S0 = 10112
H_N0 = 5120
H_N1 = 2560

P_ROWS = 10112
A0_ROWS = 5120
H_ROWS = 5120
A1_ROWS = 2560

def _mesh():
    return plsc.VectorSubcoreMesh(core_axis_name="c", subcore_axis_name="s",
                                  num_cores=NC, num_subcores=NS)


def _sc_params(vector_ops=False):
    cp = pltpu.CompilerParams(use_tc_tiling_on_sc=False)
    if vector_ops and "needs_layout_passes" in pltpu.CompilerParams.__dataclass_fields__:
        cp = dataclasses.replace(cp, needs_layout_passes=False)
    return cp
_f32 = jnp.float32


def _zero_rows(z_v, n):
    @pl.loop(0, n)
    def _(i):
        z_v[i, :] = jnp.zeros((16,), _f32)


def _hist_body(s0, d0, s1, d1, o0, o1, o2, o3,
               g0, g1, g2, g3, idx_v, sm0, sm1, sm2, sm3):
    c = lax.axis_index("c")
    s = lax.axis_index("s")
    wid = s * NC + c
    ones = jnp.ones((16,), _f32)

    segs = ((s0, g0, H_S0, NCH0, 0, o0, sm0),
            (d0, g1, H_N0, NCH0, NCH0, o1, sm1),
            (s1, g2, H_N0, NCH1, 2 * NCH0, o2, sm2),
            (d1, g3, H_N1, NCH1, 2 * NCH0 + NCH1, o3, sm3))
    for arr, _, _, nch, off, _, sm in segs:
        pltpu.async_copy(arr.at[wid], idx_v.at[pl.ds(off, nch)], sm)

    for arr, hv, hn, nch, off, o, sm in segs:
        @pl.loop(0, hn, step=128)
        def _(i, hv=hv):
            for k in range(8):
                hv[pl.ds(i + k * 16, 16)] = jnp.zeros((16,), _f32)

        pltpu.make_async_copy(arr.at[wid], idx_v.at[pl.ds(off, nch)], sm).wait()

        @pl.loop(off, off + nch, unroll=2)
        def _(j, hv=hv):
            for k in range(CH // 16):
                plsc.addupdate_scatter(hv, [idx_v[j, pl.ds(k * 16, 16)]], ones)

        pltpu.async_copy(hv, o.at[wid], sm)
    for arr, hv, hn, nch, off, o, sm in segs:
        pltpu.make_async_copy(hv, o.at[wid], sm).wait()


@jax.jit
def _hist(s0p, d0p, s1p, d1p):
    f = pl.kernel(
        _hist_body,
        out_type=(jax.ShapeDtypeStruct((NW, H_S0), _f32),
                  jax.ShapeDtypeStruct((NW, H_N0), _f32),
                  jax.ShapeDtypeStruct((NW, H_N0), _f32),
                  jax.ShapeDtypeStruct((NW, H_N1), _f32)),
        mesh=_mesh(),
        compiler_params=_sc_params(vector_ops=True),
        scratch_types=[
            pltpu.VMEM((H_S0,), _f32),
            pltpu.VMEM((H_N0,), _f32),
            pltpu.VMEM((H_N0,), _f32),
            pltpu.VMEM((H_N1,), _f32),
            pltpu.VMEM((2 * (NCH0 + NCH1), CH), jnp.int32),
            pltpu.SemaphoreType.DMA,
            pltpu.SemaphoreType.DMA,
            pltpu.SemaphoreType.DMA,
            pltpu.SemaphoreType.DMA,
        ],
    )
    return f(s0p, d0p, s1p, d1p)


def _edge_body(nch, tab_rows, acc_rows,
               feat, src, dst, out, tab_sh, acc_sh, src_v, dst_v,
               rows0, rows1, z_v, gs0, gs1, ss0, ss1):
    c = lax.axis_index("c")
    s = lax.axis_index("s")
    wid = s * NC + c
    kt = tab_rows // 16
    ka = acc_rows // 16

    _zero_rows(z_v, ka)
    pltpu.async_copy(feat.at[pl.ds(s * kt, kt)], tab_sh.at[pl.ds(s * kt, kt)], gs0)
    pltpu.async_copy(z_v.at[pl.ds(0, ka)], acc_sh.at[pl.ds(s * ka, ka)], gs1)
    pltpu.async_copy(src.at[wid], src_v, ss0)
    pltpu.async_copy(dst.at[wid], dst_v, ss1)
    pltpu.make_async_copy(feat.at[pl.ds(s * kt, kt)], tab_sh.at[pl.ds(s * kt, kt)], gs0).wait()
    pltpu.make_async_copy(z_v.at[pl.ds(0, ka)], acc_sh.at[pl.ds(s * ka, ka)], gs1).wait()
    pltpu.make_async_copy(src.at[wid], src_v, ss0).wait()
    pltpu.make_async_copy(dst.at[wid], dst_v, ss1).wait()
    plsc.subcore_barrier()

    pltpu.async_copy(tab_sh.at[src_v.at[0]], rows0, gs0)
    pltpu.async_copy(tab_sh.at[src_v.at[1]], rows1, gs1)

    @pl.loop(0, nch, step=2)
    def _(j):
        pltpu.make_async_copy(tab_sh.at[src_v.at[0]], rows0, gs0).wait()
        pltpu.async_copy(rows0, acc_sh.at[dst_v.at[j]], ss0, add=True)
        pltpu.make_async_copy(tab_sh.at[src_v.at[0]], rows1, gs1).wait()
        pltpu.async_copy(rows1, acc_sh.at[dst_v.at[j + 1]], ss1, add=True)
        pltpu.make_async_copy(rows0, acc_sh.at[dst_v.at[0]], ss0).wait()

        @pl.when(j + 2 < nch)
        def _():
            pltpu.async_copy(tab_sh.at[src_v.at[j + 2]], rows0, gs0)

        pltpu.make_async_copy(rows1, acc_sh.at[dst_v.at[0]], ss1).wait()

        @pl.when(j + 3 < nch)
        def _():
            pltpu.async_copy(tab_sh.at[src_v.at[j + 3]], rows1, gs1)

    plsc.subcore_barrier()
    pltpu.sync_copy(acc_sh.at[pl.ds(s * ka, ka)], out.at[c].at[pl.ds(s * ka, ka)])


def _make_edge(nch, tab_rows, acc_rows):
    @jax.jit
    def f(feat, src, dst):
        body = functools.partial(_edge_body, nch, tab_rows, acc_rows)
        k = pl.kernel(
            body,
            out_type=jax.ShapeDtypeStruct((NC, acc_rows, D_HID), _f32),
            mesh=_mesh(),
            compiler_params=_sc_params(),
            scratch_types=[
                pltpu.VMEM_SHARED((tab_rows, D_HID), _f32),
                pltpu.VMEM_SHARED((acc_rows, D_HID), _f32),
                pltpu.VMEM((nch, CH), jnp.int32),
                pltpu.VMEM((nch, CH), jnp.int32),
                pltpu.VMEM((CH, D_HID), _f32),
                pltpu.VMEM((CH, D_HID), _f32),
                pltpu.VMEM((acc_rows // 16, D_HID), _f32),
                pltpu.SemaphoreType.DMA,
                pltpu.SemaphoreType.DMA,
                pltpu.SemaphoreType.DMA,
                pltpu.SemaphoreType.DMA,
            ],
        )
        return k(feat, src, dst)
    return f


_edge0 = _make_edge(NCH0, P_ROWS, A0_ROWS)
_edge1 = _make_edge(NCH1, H_ROWS, A1_ROWS)


def _mm_body(x_ref, w_ref, o_ref):
    o_ref[...] = jnp.dot(x_ref[...], w_ref[...], preferred_element_type=_f32)


@jax.jit
def _mm(x, w):
    return pl.pallas_call(
        _mm_body,
        out_shape=jax.ShapeDtypeStruct((N_NODES, D_HID), _f32),
    )(x, w)


def _scale_body(d_ref, p_ref, o_ref):
    d = jnp.sum(d_ref[...], axis=0)[:N_NODES]
    r = lax.rsqrt(jnp.maximum(d, 1.0))
    o_ref[pl.ds(0, N_NODES), :] = p_ref[...] * r[:, None]
    o_ref[pl.ds(N_NODES, P_ROWS - N_NODES), :] = jnp.zeros(
        (P_ROWS - N_NODES, D_HID), _f32)


@jax.jit
def _proj(deg, p0):
    return pl.pallas_call(
        _scale_body,
        out_shape=jax.ShapeDtypeStruct((P_ROWS, D_HID), _f32),
    )(deg, p0)


def _mid_body(p_ref, din_ref, dout_ref, b_ref, o_ref):
    h = p_ref[0, :N_DST0] + p_ref[1, :N_DST0]
    rin = lax.rsqrt(jnp.maximum(jnp.sum(din_ref[...], axis=0)[:N_DST0], 1.0))
    rout = lax.rsqrt(jnp.maximum(jnp.sum(dout_ref[...], axis=0)[:N_DST0], 1.0))
    h = jnp.maximum(h * rin[:, None] + b_ref[...], 0.0) * rout[:, None]
    o_ref[pl.ds(0, N_DST0), :] = h
    o_ref[pl.ds(N_DST0, H_ROWS - N_DST0), :] = jnp.zeros(
        (H_ROWS - N_DST0, D_HID), _f32)


@jax.jit
def _mid(p, din, dout, b):
    return pl.pallas_call(
        _mid_body,
        out_shape=jax.ShapeDtypeStruct((H_ROWS, D_HID), _f32),
    )(p, din, dout, b)


def _out_body(q_ref, din_ref, w_ref, b_ref, o_ref):
    agg = q_ref[0, :N_DST1] + q_ref[1, :N_DST1]
    r = lax.rsqrt(jnp.maximum(jnp.sum(din_ref[...], axis=0)[:N_DST1], 1.0))
    o_ref[...] = jnp.dot(agg, w_ref[...],
                         preferred_element_type=_f32) * r[:, None] + b_ref[...]


@jax.jit
def _outk(q, din, w, b):
    return pl.pallas_call(
        _out_body,
        out_shape=jax.ShapeDtypeStruct((N_DST1, D_OUT), _f32),
    )(q, din, w, b)


def kernel(x, src0, dst0, src1, dst1, W1, b1, W2, b2):
    i32 = jnp.int32
    src0 = src0.astype(i32)
    dst0 = dst0.astype(i32)
    src1 = src1.astype(i32)
    dst1 = dst1.astype(i32)

    pad0 = NW * NCH0 * CH - E0
    s0p = jnp.concatenate([src0, jnp.full((pad0,), N_NODES, i32)]).reshape(NW, NCH0, CH)
    d0p = jnp.concatenate([dst0, jnp.full((pad0,), N_DST0, i32)]).reshape(NW, NCH0, CH)
    pad1 = NW * NCH1 * CH - E1
    s1p = jnp.concatenate([src1, jnp.full((pad1,), N_DST0, i32)]).reshape(NW, NCH1, CH)
    d1p = jnp.concatenate([dst1, jnp.full((pad1,), N_DST1, i32)]).reshape(NW, NCH1, CH)

    p0 = _mm(x, W1)
    h_s0, h_d0, h_s1, h_d1 = _hist(s0p, d0p, s1p, d1p)

    p = _proj(h_s0, p0)
    agg0 = _edge0(p, s0p, d0p)

    hmid = _mid(agg0, h_d0, h_s1, b1.reshape(1, D_HID))
    agg1 = _edge1(hmid, s1p, d1p)

    return _outk(agg1, h_d1, W2, b2.reshape(1, D_OUT))
